# baseline (speedup 1.0000x reference)
import functools

import jax
import jax.numpy as jnp
from jax import lax
from jax.experimental import pallas as pl
from jax.experimental.pallas import tpu as pltpu
from jax.experimental.pallas import tpu_sc as plsc

NN = 10000
EE = 160000
G = 64
DIN = 768
DH = 128
DOUT = 20

NP = 10240
EP = 163840
CH = 128
NC = 2
NS = 16
NW = NC * NS
CPW = EP // CH // NW
RPS = NP // NS

BPW = NP // NW
NBKT = NW + 1
CAPC = 22 * CH
BINW = NBKT * CAPC
TRASH = BPW

_sc_mesh = plsc.VectorSubcoreMesh(core_axis_name="c", subcore_axis_name="s",
                                  num_cores=NC, num_subcores=NS)


@functools.partial(
    pl.kernel,
    out_type=jax.ShapeDtypeStruct((NC, NP), jnp.float32),
    mesh=_sc_mesh,
    scratch_types=[
        pltpu.VMEM((CPW, CH), jnp.int32),
        pltpu.VMEM((CH,), jnp.float32),
        pltpu.VMEM_SHARED((NP,), jnp.float32),
    ],
)
def _deg_kernel(colp2, zeros1, out, cidx, ones_v, deg_sh):
    c = lax.axis_index("c")
    s = lax.axis_index("s")
    w = s * NC + c
    for i in range(CH // 16):
        ones_v[pl.ds(i * 16, 16)] = jnp.ones((16,), jnp.float32)
    pltpu.sync_copy(zeros1.at[pl.ds(s * RPS, RPS)],
                    deg_sh.at[pl.ds(s * RPS, RPS)])
    pltpu.sync_copy(colp2.at[pl.ds(w * CPW, CPW)], cidx)
    plsc.subcore_barrier()

    def body(k, carry):
        pltpu.sync_copy(ones_v, deg_sh.at[cidx.at[k]], add=True)
        return carry

    lax.fori_loop(0, CPW, body, 0)
    plsc.subcore_barrier()
    pltpu.sync_copy(deg_sh.at[pl.ds(s * RPS, RPS)],
                    out.at[c, pl.ds(s * RPS, RPS)])


_RWS = 40


def _binpos_body(bktb, posb, carry):
    i = pl.program_id(0)

    @pl.when(i == 0)
    def _():
        carry[...] = jnp.zeros_like(carry)

    cc0 = (i % 2) == 0
    ccoff = jnp.where(cc0, 0, BINW)
    iota_n = lax.broadcasted_iota(jnp.int32, (CH, CH), 0)
    iota_e = lax.broadcasted_iota(jnp.int32, (CH, CH), 1)
    trilT = (iota_n < iota_e).astype(jnp.bfloat16)
    for s in range(_RWS):
        brow = bktb[s:s + 1, :]
        M = jnp.broadcast_to(brow, (CH, CH)) == iota_n
        Mb = M.astype(jnp.bfloat16)
        before = lax.dot_general(Mb, trilT, (((1,), (0,)), ((), ())),
                                 preferred_element_type=jnp.float32)
        rank = jnp.sum(jnp.where(M, before, 0.0), axis=0, keepdims=True)
        c0 = carry[:, 0:1]
        c1 = carry[:, 1:2]
        ccol = jnp.where(cc0, c0, c1)
        cgat = jnp.sum(jnp.where(M, jnp.broadcast_to(ccol, (CH, CH)), 0.0),
                       axis=0, keepdims=True)
        cnts = jnp.sum(Mb.astype(jnp.float32), axis=1, keepdims=True)
        cnew = ccol + cnts
        carry[:, 0:1] = jnp.where(cc0, cnew, c0)
        carry[:, 1:2] = jnp.where(cc0, c1, cnew)
        crank = (rank + cgat).astype(jnp.int32)
        pos = ccoff + brow * CAPC + crank
        pos = jnp.where(crank < CAPC, pos, ccoff + NW * CAPC + CAPC - 1)
        posb[s:s + 1, :] = pos


def _binpos_call(bkt2):
    return pl.pallas_call(
        _binpos_body,
        grid=(EP // CH // _RWS,),
        in_specs=[pl.BlockSpec((_RWS, CH), lambda i: (i, 0))],
        out_specs=pl.BlockSpec((_RWS, CH), lambda i: (i, 0)),
        out_shape=jax.ShapeDtypeStruct((EP // CH, CH), jnp.int32),
        scratch_shapes=[pltpu.VMEM((CH, NC), jnp.float32)],
    )(bkt2)


_FILLB = 704


@functools.partial(
    pl.kernel,
    out_type=[
        jax.ShapeDtypeStruct((NC * BINW,), jnp.int32),
        jax.ShapeDtypeStruct((NC * BINW,), jnp.int32),
    ],
    mesh=_sc_mesh,
    scratch_types=[
        pltpu.VMEM((CPW, CH), jnp.int32),
        pltpu.VMEM((CPW, CH), jnp.int32),
        pltpu.VMEM((CPW, CH), jnp.int32),
        pltpu.VMEM((_FILLB,), jnp.int32),
        pltpu.VMEM((_FILLB,), jnp.int32),
    ],
)
def _binscat_kernel(rowp2, collp2, posp2, binrow, bincol, ridx, cidx, pidx,
                    tr_row, tr_col):
    c = lax.axis_index("c")
    s = lax.axis_index("s")
    w = s * NC + c
    for i in range(_FILLB // 16):
        tr_row[pl.ds(i * 16, 16)] = jnp.zeros((16,), jnp.int32)
        tr_col[pl.ds(i * 16, 16)] = jnp.full((16,), TRASH, jnp.int32)
    for j in range(2):
        for t in range(CAPC // _FILLB):
            off = c * BINW + (2 * s + j) * CAPC + t * _FILLB
            pltpu.sync_copy(tr_row, binrow.at[pl.ds(off, _FILLB)])
            pltpu.sync_copy(tr_col, bincol.at[pl.ds(off, _FILLB)])
    pltpu.sync_copy(rowp2.at[pl.ds(w * CPW, CPW)], ridx)
    pltpu.sync_copy(collp2.at[pl.ds(w * CPW, CPW)], cidx)
    pltpu.sync_copy(posp2.at[pl.ds(w * CPW, CPW)], pidx)
    plsc.subcore_barrier()

    def body(k, carry):
        pltpu.sync_copy(ridx.at[k], binrow.at[pidx.at[k]])
        pltpu.sync_copy(cidx.at[k], bincol.at[pidx.at[k]])
        return carry

    lax.fori_loop(0, CPW, body, 0)


NCHB = NC * (CAPC // CH)
NBUF = 4


@functools.partial(
    pl.kernel,
    out_type=jax.ShapeDtypeStruct((NP, DH), jnp.float32),
    mesh=_sc_mesh,
    scratch_types=[
        pltpu.VMEM((NCHB, CH), jnp.int32),
        pltpu.VMEM((NCHB, CH), jnp.int32),
        pltpu.VMEM((NBUF, CH, DH), jnp.float32),
        pltpu.VMEM((BPW + 8, DH), jnp.float32),
        [pltpu.SemaphoreType.DMA] * NBUF,
    ],
)
def _agg_kernel(hs, binrow3, bincol3, zeros2, out, ridx, cidx, rows_v, acc_v,
                gsems):
    c = lax.axis_index("c")
    s = lax.axis_index("s")
    w = s * NC + c
    pltpu.sync_copy(zeros2.at[pl.ds(0, BPW + 8)], acc_v)
    for c2 in range(NC):
        pltpu.sync_copy(binrow3.at[c2 * NBKT + w],
                        ridx.at[pl.ds(c2 * (CAPC // CH), CAPC // CH)])
        pltpu.sync_copy(bincol3.at[c2 * NBKT + w],
                        cidx.at[pl.ds(c2 * (CAPC // CH), CAPC // CH)])

    for b in range(NBUF):
        pltpu.async_copy(hs.at[ridx.at[b]], rows_v.at[b], gsems[b])

    def group(g, carry):
        for b in range(NBUF):
            k = g * NBUF + b
            pltpu.make_async_copy(hs.at[ridx.at[k]], rows_v.at[b],
                                  gsems[b]).wait()

            def edge16(e16, carry2):
                cl16 = cidx[k, pl.ds(e16 * 16, 16)]
                for l in range(16):
                    cl = cl16[l]
                    e = e16 * 16 + l
                    for j in range(DH // 16):
                        plsc.addupdate(acc_v.at[cl, pl.ds(j * 16, 16)],
                                       rows_v[b, e, pl.ds(j * 16, 16)])
                return carry2

            lax.fori_loop(0, CH // 16, edge16, 0)

            @pl.when(k + NBUF < NCHB)
            def _():
                pltpu.async_copy(hs.at[ridx.at[k + NBUF]], rows_v.at[b],
                                 gsems[b])
        return carry

    lax.fori_loop(0, NCHB // NBUF, group, 0)
    pltpu.sync_copy(acc_v.at[pl.ds(0, BPW)], out.at[pl.ds(w * BPW, BPW)])


_RB = 1000


def _mm1_body(xb, dgb, W1b, hsb, dvb):
    dv = lax.rsqrt(dgb[:, 0:1] + dgb[:, 1:2] + 1.0)
    h = jnp.dot(xb[...], W1b[...], preferred_element_type=jnp.float32)
    hsb[...] = h * dv
    dvb[...] = dv


def _mm1_call(x, degp_t, W1):
    return pl.pallas_call(
        _mm1_body,
        grid=(NN // _RB,),
        in_specs=[
            pl.BlockSpec((_RB, DIN), lambda i: (i, 0)),
            pl.BlockSpec((_RB, NC), lambda i: (i, 0)),
            pl.BlockSpec((DIN, DH), lambda i: (0, 0)),
        ],
        out_specs=[
            pl.BlockSpec((_RB, DH), lambda i: (i, 0)),
            pl.BlockSpec((_RB, 1), lambda i: (i, 0)),
        ],
        out_shape=[
            jax.ShapeDtypeStruct((NN, DH), jnp.float32),
            jax.ShapeDtypeStruct((NN, 1), jnp.float32),
        ],
    )(x, degp_t, W1)


def _mm2_body(pb, hsb, dvb, b1b, W2b, outb):
    t = jnp.maximum((pb[...] + hsb[...]) * dvb[...] + b1b[...], 0.0)
    outb[...] = jnp.dot(t, W2b[...], preferred_element_type=jnp.float32) * dvb[...]


def _mm2_call(p1, h1s, dinv, b1, W2):
    return pl.pallas_call(
        _mm2_body,
        grid=(NN // _RB,),
        in_specs=[
            pl.BlockSpec((_RB, DH), lambda i: (i, 0)),
            pl.BlockSpec((_RB, DH), lambda i: (i, 0)),
            pl.BlockSpec((_RB, 1), lambda i: (i, 0)),
            pl.BlockSpec((1, DH), lambda i: (0, 0)),
            pl.BlockSpec((DH, DH), lambda i: (0, 0)),
        ],
        out_specs=pl.BlockSpec((_RB, DH), lambda i: (i, 0)),
        out_shape=jax.ShapeDtypeStruct((NN, DH), jnp.float32),
    )(p1, h1s, dinv, b1, W2)


def _pool_body(pb, hsb, dvb, b2b, batchb, Wcb, bcb, outb, acc_s, acc_c):
    i = pl.program_id(0)

    @pl.when(i == 0)
    def _():
        acc_s[...] = jnp.zeros_like(acc_s)
        acc_c[...] = jnp.zeros_like(acc_c)

    h3 = jnp.maximum((pb[...] + hsb[...]) * dvb[...] + b2b[...], 0.0)
    oh = (batchb[...] == lax.broadcasted_iota(jnp.int32, (1, G), 1))
    oh = oh.astype(jnp.float32)
    acc_s[...] += lax.dot_general(oh, h3, (((0,), (0,)), ((), ())),
                                  preferred_element_type=jnp.float32)
    ones_col = jnp.ones((_RB, 1), jnp.float32)
    acc_c[...] += lax.dot_general(oh, ones_col, (((0,), (0,)), ((), ())),
                                  preferred_element_type=jnp.float32)

    @pl.when(i == pl.num_programs(0) - 1)
    def _():
        pooled = acc_s[...] / jnp.maximum(acc_c[...], 1.0)
        outb[...] = jnp.dot(pooled, Wcb[...],
                            preferred_element_type=jnp.float32) + bcb[...]


def _pool_call(p2, h2s, dinv, b2, batch2d, Wc, bc):
    return pl.pallas_call(
        _pool_body,
        grid=(NN // _RB,),
        in_specs=[
            pl.BlockSpec((_RB, DH), lambda i: (i, 0)),
            pl.BlockSpec((_RB, DH), lambda i: (i, 0)),
            pl.BlockSpec((_RB, 1), lambda i: (i, 0)),
            pl.BlockSpec((1, DH), lambda i: (0, 0)),
            pl.BlockSpec((_RB, 1), lambda i: (i, 0)),
            pl.BlockSpec((DH, DOUT), lambda i: (0, 0)),
            pl.BlockSpec((1, DOUT), lambda i: (0, 0)),
        ],
        out_specs=pl.BlockSpec((G, DOUT), lambda i: (0, 0)),
        out_shape=jax.ShapeDtypeStruct((G, DOUT), jnp.float32),
        scratch_shapes=[
            pltpu.VMEM((G, DH), jnp.float32),
            pltpu.VMEM((G, 1), jnp.float32),
        ],
    )(p2, h2s, dinv, b2, batch2d, Wc, bc)


def kernel(x, edge_index, batch, W1, b1, W2, b2, Wc, bc):
    row = edge_index[0]
    col = edge_index[1]
    pad = EP - EE
    rowp1 = jnp.concatenate([row, jnp.zeros((pad,), row.dtype)])
    colp1 = jnp.concatenate([col, jnp.full((pad,), NP, col.dtype)])
    bkt1 = colp1 // BPW
    coll1 = colp1 - bkt1 * BPW
    rowp = rowp1.reshape(EP // CH, CH)
    colp_d = jnp.concatenate(
        [col, jnp.full((pad,), NP - 1, col.dtype)]).reshape(EP // CH, CH)
    bkt2 = bkt1.reshape(EP // CH, CH)
    coll2 = coll1.reshape(EP // CH, CH)
    zeros1 = jnp.zeros((NP,), jnp.float32)
    zeros2 = jnp.zeros((NP, DH), jnp.float32)

    degp = _deg_kernel(colp_d, zeros1)
    degp_t = jnp.transpose(degp)[:NN]

    pos2 = _binpos_call(bkt2)
    binrow, bincol = _binscat_kernel(rowp, coll2, pos2)
    binrow3 = binrow.reshape(NC * NBKT, CAPC // CH, CH)
    bincol3 = bincol.reshape(NC * NBKT, CAPC // CH, CH)

    h1s, dinv = _mm1_call(x, degp_t, W1)
    p1 = _agg_kernel(h1s, binrow3, bincol3, zeros2)
    h2s = _mm2_call(p1, h1s, dinv, b1.reshape(1, -1), W2)
    p2 = _agg_kernel(h2s, binrow3, bincol3, zeros2)
    out = _pool_call(p2, h2s, dinv, b2.reshape(1, -1),
                     batch.reshape(-1, 1), Wc, bc.reshape(1, -1))
    return out.reshape(-1)


# speedup vs baseline: 4.2245x; 4.2245x over previous
import functools

import jax
import jax.numpy as jnp
from jax import lax
from jax.experimental import pallas as pl
from jax.experimental.pallas import tpu as pltpu
from jax.experimental.pallas import tpu_sc as plsc

NN = 10000
EE = 160000
G = 64
DIN = 768
DH = 128
DOUT = 20

NP = 10240
EP = 163840
CH = 128
NC = 2
NS = 16
NW = NC * NS
CPW = EP // CH // NW
RPS = NP // NS

_sc_mesh = plsc.VectorSubcoreMesh(core_axis_name="c", subcore_axis_name="s",
                                  num_cores=NC, num_subcores=NS)


@functools.partial(
    pl.kernel,
    out_type=jax.ShapeDtypeStruct((NC, NP), jnp.float32),
    mesh=_sc_mesh,
    scratch_types=[
        pltpu.VMEM((CPW, CH), jnp.int32),
        pltpu.VMEM((CH,), jnp.float32),
        pltpu.VMEM_SHARED((NP,), jnp.float32),
    ],
)
def _deg_kernel(colp2, zeros1, out, cidx, ones_v, deg_sh):
    c = lax.axis_index("c")
    s = lax.axis_index("s")
    w = s * NC + c
    for i in range(CH // 16):
        ones_v[pl.ds(i * 16, 16)] = jnp.ones((16,), jnp.float32)
    pltpu.sync_copy(zeros1.at[pl.ds(s * RPS, RPS)],
                    deg_sh.at[pl.ds(s * RPS, RPS)])
    pltpu.sync_copy(colp2.at[pl.ds(w * CPW, CPW)], cidx)
    plsc.subcore_barrier()

    def body(k, carry):
        pltpu.sync_copy(ones_v, deg_sh.at[cidx.at[k]], add=True)
        return carry

    lax.fori_loop(0, CPW, body, 0)
    plsc.subcore_barrier()
    pltpu.sync_copy(deg_sh.at[pl.ds(s * RPS, RPS)],
                    out.at[c, pl.ds(s * RPS, RPS)])


CHA = 128
CPWA = EP // CHA // NW
NBUF = 2
NG = CPWA // NBUF


@functools.partial(
    pl.kernel,
    out_type=jax.ShapeDtypeStruct((NC, NP, DH), jnp.float32),
    mesh=_sc_mesh,
    scratch_types=[
        pltpu.VMEM((CPWA, CHA), jnp.int32),
        pltpu.VMEM((CPWA, CHA), jnp.int32),
        pltpu.VMEM((NBUF, CHA, DH), jnp.float32),
        pltpu.VMEM_SHARED((NP, DH), jnp.float32),
        [pltpu.SemaphoreType.DMA] * NBUF,
        [pltpu.SemaphoreType.DMA] * NBUF,
    ],
)
def _agg_kernel(hs, rowp2, colp2, zeros2, out, ridx, cidx, rows_v, acc_sh,
                gsems, ssems):
    c = lax.axis_index("c")
    s = lax.axis_index("s")
    w = s * NC + c
    pltpu.sync_copy(zeros2.at[pl.ds(s * RPS, RPS)],
                    acc_sh.at[pl.ds(s * RPS, RPS)])
    pltpu.sync_copy(rowp2.at[pl.ds(w * CPWA, CPWA)], ridx)
    pltpu.sync_copy(colp2.at[pl.ds(w * CPWA, CPWA)], cidx)
    plsc.subcore_barrier()

    for b in range(NBUF):
        pltpu.async_copy(hs.at[ridx.at[b]], rows_v.at[b], gsems[b])

    def group(g, carry):
        for b in range(NBUF):
            k = g * NBUF + b
            pltpu.make_async_copy(hs.at[ridx.at[k]], rows_v.at[b],
                                  gsems[b]).wait()
            pltpu.sync_copy(rows_v.at[b], acc_sh.at[cidx.at[k]], add=True)

            @pl.when(k + NBUF < CPWA)
            def _():
                pltpu.async_copy(hs.at[ridx.at[k + NBUF]], rows_v.at[b],
                                 gsems[b])
        return carry

    lax.fori_loop(0, NG, group, 0)
    plsc.subcore_barrier()
    pltpu.sync_copy(acc_sh.at[pl.ds(s * RPS, RPS)],
                    out.at[c, pl.ds(s * RPS, RPS)])


_RB = 1000


def _mm1_body(xb, dgb, W1b, hsb, dvb):
    dv = lax.rsqrt(dgb[:, 0:1] + dgb[:, 1:2] + 1.0)
    h = jnp.dot(xb[...], W1b[...], preferred_element_type=jnp.float32)
    hsb[...] = h * dv
    dvb[...] = dv


def _mm1_call(x, degp_t, W1):
    return pl.pallas_call(
        _mm1_body,
        grid=(NN // _RB,),
        in_specs=[
            pl.BlockSpec((_RB, DIN), lambda i: (i, 0)),
            pl.BlockSpec((_RB, NC), lambda i: (i, 0)),
            pl.BlockSpec((DIN, DH), lambda i: (0, 0)),
        ],
        out_specs=[
            pl.BlockSpec((_RB, DH), lambda i: (i, 0)),
            pl.BlockSpec((_RB, 1), lambda i: (i, 0)),
        ],
        out_shape=[
            jax.ShapeDtypeStruct((NN, DH), jnp.float32),
            jax.ShapeDtypeStruct((NN, 1), jnp.float32),
        ],
    )(x, degp_t, W1)


def _mm2_body(pb, hsb, dvb, b1b, W2b, outb):
    t = jnp.maximum((pb[0] + pb[1] + hsb[...]) * dvb[...] + b1b[...], 0.0)
    outb[...] = jnp.dot(t, W2b[...], preferred_element_type=jnp.float32) * dvb[...]


def _mm2_call(p1, h1s, dinv, b1, W2):
    return pl.pallas_call(
        _mm2_body,
        grid=(NN // _RB,),
        in_specs=[
            pl.BlockSpec((NC, _RB, DH), lambda i: (0, i, 0)),
            pl.BlockSpec((_RB, DH), lambda i: (i, 0)),
            pl.BlockSpec((_RB, 1), lambda i: (i, 0)),
            pl.BlockSpec((1, DH), lambda i: (0, 0)),
            pl.BlockSpec((DH, DH), lambda i: (0, 0)),
        ],
        out_specs=pl.BlockSpec((_RB, DH), lambda i: (i, 0)),
        out_shape=jax.ShapeDtypeStruct((NN, DH), jnp.float32),
    )(p1, h1s, dinv, b1, W2)


def _pool_body(pb, hsb, dvb, b2b, batchb, Wcb, bcb, outb, acc_s, acc_c):
    i = pl.program_id(0)

    @pl.when(i == 0)
    def _():
        acc_s[...] = jnp.zeros_like(acc_s)
        acc_c[...] = jnp.zeros_like(acc_c)

    h3 = jnp.maximum((pb[0] + pb[1] + hsb[...]) * dvb[...] + b2b[...], 0.0)
    oh = (batchb[...] == lax.broadcasted_iota(jnp.int32, (1, G), 1))
    oh = oh.astype(jnp.float32)
    acc_s[...] += lax.dot_general(oh, h3, (((0,), (0,)), ((), ())),
                                  preferred_element_type=jnp.float32)
    ones_col = jnp.ones((_RB, 1), jnp.float32)
    acc_c[...] += lax.dot_general(oh, ones_col, (((0,), (0,)), ((), ())),
                                  preferred_element_type=jnp.float32)

    @pl.when(i == pl.num_programs(0) - 1)
    def _():
        pooled = acc_s[...] / jnp.maximum(acc_c[...], 1.0)
        outb[...] = jnp.dot(pooled, Wcb[...],
                            preferred_element_type=jnp.float32) + bcb[...]


def _pool_call(p2, h2s, dinv, b2, batch2d, Wc, bc):
    return pl.pallas_call(
        _pool_body,
        grid=(NN // _RB,),
        in_specs=[
            pl.BlockSpec((NC, _RB, DH), lambda i: (0, i, 0)),
            pl.BlockSpec((_RB, DH), lambda i: (i, 0)),
            pl.BlockSpec((_RB, 1), lambda i: (i, 0)),
            pl.BlockSpec((1, DH), lambda i: (0, 0)),
            pl.BlockSpec((_RB, 1), lambda i: (i, 0)),
            pl.BlockSpec((DH, DOUT), lambda i: (0, 0)),
            pl.BlockSpec((1, DOUT), lambda i: (0, 0)),
        ],
        out_specs=pl.BlockSpec((G, DOUT), lambda i: (0, 0)),
        out_shape=jax.ShapeDtypeStruct((G, DOUT), jnp.float32),
        scratch_shapes=[
            pltpu.VMEM((G, DH), jnp.float32),
            pltpu.VMEM((G, 1), jnp.float32),
        ],
    )(p2, h2s, dinv, b2, batch2d, Wc, bc)


def kernel(x, edge_index, batch, W1, b1, W2, b2, Wc, bc):
    row = edge_index[0]
    col = edge_index[1]
    pad = EP - EE
    rowp1 = jnp.concatenate([row, jnp.zeros((pad,), row.dtype)])
    colp1 = jnp.concatenate([col, jnp.full((pad,), NP - 1, col.dtype)])
    rowp = rowp1.reshape(EP // CHA, CHA)
    colp = colp1.reshape(EP // CHA, CHA)
    colp_d = colp1.reshape(EP // CH, CH)
    zeros1 = jnp.zeros((NP,), jnp.float32)
    zeros2 = jnp.zeros((NP, DH), jnp.float32)

    degp = _deg_kernel(colp_d, zeros1)
    degp_t = jnp.transpose(degp)[:NN]

    h1s, dinv = _mm1_call(x, degp_t, W1)
    p1 = _agg_kernel(h1s, rowp, colp, zeros2)
    h2s = _mm2_call(p1, h1s, dinv, b1.reshape(1, -1), W2)
    p2 = _agg_kernel(h2s, rowp, colp, zeros2)
    out = _pool_call(p2, h2s, dinv, b2.reshape(1, -1),
                     batch.reshape(-1, 1), Wc, bc.reshape(1, -1))
    return out.reshape(-1)
